# baseline (speedup 1.0000x reference)
import jax
import jax.numpy as jnp
from jax import lax
from jax.experimental import pallas as pl
from jax.experimental.pallas import tpu as pltpu
from jax.experimental.pallas import tpu_sc as plsc

_B = 16384
_D = 32
_NC = 2
_NS = 16
_NW = _NC * _NS
_BPW = _B // _NW
_CH = 128
_NCH = _BPW // _CH

_BN = 2048
_NB = _B // _BN


_W1R = 1024
_W2R = 100000
_W2PW = 3128


def _sc_repack_body(w1, w2, w1p, w2p, sem):
    wid = lax.axis_index("s") * _NC + lax.axis_index("c")

    @pl.when(wid < 31)
    def _():
        pltpu.async_copy(w2.at[pl.ds(wid * _W2PW, _W2PW)],
                         w2p.at[pl.ds(wid * _W2PW, _W2PW), pl.ds(0, _D)],
                         sem).wait()

    @pl.when(wid == 31)
    def _():
        pltpu.async_copy(w2.at[pl.ds(31 * _W2PW, _W2R - 31 * _W2PW)],
                         w2p.at[pl.ds(31 * _W2PW, _W2R - 31 * _W2PW), pl.ds(0, _D)],
                         sem).wait()

    @pl.when(wid < 31)
    def _():
        pltpu.async_copy(w1.at[pl.ds(wid * 32, 32)],
                         w1p.at[pl.ds(wid * 32, 32), pl.ds(0, _D)], sem).wait()

    @pl.when(wid == 31)
    def _():
        pltpu.async_copy(w1.at[pl.ds(992, 8)],
                         w1p.at[pl.ds(992, 8), pl.ds(0, _D)], sem).wait()


def _sc_gather_body(w1, w2, idx1, idx2, emb, idx1_v, idx2_v, rows_v, sem):
    wid = lax.axis_index("s") * _NC + lax.axis_index("c")
    base = wid * _BPW
    pltpu.sync_copy(idx1.at[pl.ds(base, _BPW)], idx1_v)
    pltpu.sync_copy(idx2.at[pl.ds(base, _BPW)], idx2_v)
    copies = []
    for j in range(_NCH):
        copies.append(
            pltpu.async_copy(w1.at[idx1_v.at[pl.ds(j * _CH, _CH)]],
                             rows_v.at[pl.ds(j * _CH, _CH)], sem))
    for c in copies:
        c.wait()
    pltpu.sync_copy(rows_v, emb.at[0, pl.ds(base, _BPW)])
    copies = []
    for j in range(_NCH):
        copies.append(
            pltpu.async_copy(w2.at[idx2_v.at[pl.ds(j * _CH, _CH)]],
                             rows_v.at[pl.ds(j * _CH, _CH)], sem))
    for c in copies:
        c.wait()
    pltpu.sync_copy(rows_v, emb.at[1, pl.ds(base, _BPW)])


def _tc_pad_body(w_ref, out_ref):
    out_ref[:, 0:_D] = w_ref[:, :]


def _tc_mlp_body(e1_ref, e2_ref, c1_ref, c2_ref, ck_ref, lk_ref, cm_ref,
                 wc, bc, gc, bec, wl, bl, gl, bel, wm, bm, gm, bem,
                 st_ref, out_ref):
    m1 = (c1_ref[0, 0, :] != 0).astype(jnp.float32)[None, :]
    m2 = (c2_ref[0, 0, :] != 0).astype(jnp.float32)[None, :]
    out_ref[0:_D, :] = e1_ref[0, :, 0:_D].T * m1
    out_ref[_D:2 * _D, :] = e2_ref[0, :, 0:_D].T * m2

    def mlp(x, w, b, g, be, m, s):
        z = (jnp.log1p(x) - m) / s
        h = jnp.maximum(w[0][:, None] * z[None, :] + b[0][:, None], 0.0)
        mu = h.mean(0, keepdims=True)
        var = ((h - mu) ** 2).mean(0, keepdims=True)
        return (h - mu) * lax.rsqrt(var + 1e-5) * g[0][:, None] + be[0][:, None]

    out_ref[2 * _D:3 * _D, :] = mlp(ck_ref[0, 0, :], wc, bc, gc, bec,
                                    st_ref[0], st_ref[1])
    out_ref[3 * _D:4 * _D, :] = mlp(lk_ref[0, 0, :], wl, bl, gl, bel,
                                    st_ref[2], st_ref[3])
    out_ref[4 * _D:5 * _D, :] = mlp(cm_ref[0, 0, :], wm, bm, gm, bem,
                                    st_ref[4], st_ref[5])


def kernel(category_first, category_second, click_count, like_count, comment_count,
           W_cat1, W_cat2,
           w_click, b_click, g_click, be_click,
           w_like, b_like, g_like, be_like,
           w_comment, b_comment, g_comment, be_comment,
           m_click, s_click, m_like, s_like, m_comment, s_comment):
    idx1 = category_first.astype(jnp.int32)
    idx2 = category_second.astype(jnp.int32)
    w1p = jnp.pad(W_cat1, ((0, 0), (0, 128 - _D)))
    w2p = pl.pallas_call(
        _tc_pad_body,
        grid=(50,),
        in_specs=[pl.BlockSpec((_W2R // 50, _D), lambda i: (i, 0))],
        out_specs=pl.BlockSpec((_W2R // 50, 128), lambda i: (i, 0)),
        out_shape=jax.ShapeDtypeStruct((_W2R, 128), jnp.float32),
    )(W_cat2)

    sc_gather = pl.kernel(
        _sc_gather_body,
        out_type=jax.ShapeDtypeStruct((2, _B, 128), jnp.float32),
        mesh=plsc.VectorSubcoreMesh(core_axis_name="c", subcore_axis_name="s"),
        scratch_types=[
            pltpu.VMEM((_BPW,), jnp.int32),
            pltpu.VMEM((_BPW,), jnp.int32),
            pltpu.VMEM((_BPW, 128), jnp.float32),
            pltpu.SemaphoreType.DMA,
        ],
        compiler_params=pltpu.CompilerParams(use_tc_tiling_on_sc=True),
    )
    emb = sc_gather(w1p, w2p, idx1, idx2)

    stats = jnp.stack([m_click, s_click, m_like, s_like, m_comment, s_comment])

    row_spec = pl.BlockSpec((1, 1, _BN), lambda i: (i, 0, 0))
    par_spec = pl.BlockSpec((1, _D), lambda i: (0, 0))
    out = pl.pallas_call(
        _tc_mlp_body,
        grid=(_NB,),
        in_specs=[
            pl.BlockSpec((1, _BN, 128), lambda i: (0, i, 0)),
            pl.BlockSpec((1, _BN, 128), lambda i: (1, i, 0)),
            row_spec, row_spec, row_spec, row_spec, row_spec,
            par_spec, par_spec, par_spec, par_spec,
            par_spec, par_spec, par_spec, par_spec,
            par_spec, par_spec, par_spec, par_spec,
            pl.BlockSpec(memory_space=pltpu.SMEM),
        ],
        out_specs=pl.BlockSpec((5 * _D, _BN), lambda i: (0, i)),
        out_shape=jax.ShapeDtypeStruct((5 * _D, _B), jnp.float32),
    )(emb, emb,
      idx1.reshape(_NB, 1, _BN), idx2.reshape(_NB, 1, _BN),
      click_count.reshape(_NB, 1, _BN), like_count.reshape(_NB, 1, _BN),
      comment_count.reshape(_NB, 1, _BN),
      w_click, b_click.reshape(1, _D), g_click.reshape(1, _D), be_click.reshape(1, _D),
      w_like, b_like.reshape(1, _D), g_like.reshape(1, _D), be_like.reshape(1, _D),
      w_comment, b_comment.reshape(1, _D), g_comment.reshape(1, _D), be_comment.reshape(1, _D),
      stats)
    return out.T


# speedup vs baseline: 1.3357x; 1.3357x over previous
import jax
import jax.numpy as jnp
from jax import lax
from jax.experimental import pallas as pl
from jax.experimental.pallas import tpu as pltpu
from jax.experimental.pallas import tpu_sc as plsc

_B = 16384
_D = 32
_NC = 2
_NS = 16
_NW = _NC * _NS
_BPW = _B // _NW
_CH = 128
_NCH = _BPW // _CH

_BN = 2048
_NB = _B // _BN


_W2R = 100000


def _sc_gather_body(w1, w2, idx1, idx2, emb, idx1_v, idx2_v, rows_v, sem):
    wid = lax.axis_index("s") * _NC + lax.axis_index("c")
    base = wid * _BPW
    pltpu.sync_copy(idx1.at[pl.ds(base, _BPW)], idx1_v)
    pltpu.sync_copy(idx2.at[pl.ds(base, _BPW)], idx2_v)
    copies = []
    for j in range(_NCH):
        copies.append(
            pltpu.async_copy(w1.at[idx1_v.at[pl.ds(j * _CH, _CH)]],
                             rows_v.at[pl.ds(j * _CH, _CH)], sem))
    for c in copies:
        c.wait()
    pltpu.sync_copy(rows_v, emb.at[0, pl.ds(base, _BPW)])
    copies = []
    for j in range(_NCH):
        copies.append(
            pltpu.async_copy(w2.at[idx2_v.at[pl.ds(j * _CH, _CH)]],
                             rows_v.at[pl.ds(j * _CH, _CH)], sem))
    for c in copies:
        c.wait()
    pltpu.sync_copy(rows_v, emb.at[1, pl.ds(base, _BPW)])


def _tc_mlp_body(e1_ref, e2_ref, c1_ref, c2_ref, ck_ref, lk_ref, cm_ref,
                 wc, bc, gc, bec, wl, bl, gl, bel, wm, bm, gm, bem,
                 st_ref, out_ref):
    m1 = (c1_ref[0, 0, :] != 0).astype(jnp.float32)[None, :]
    m2 = (c2_ref[0, 0, :] != 0).astype(jnp.float32)[None, :]
    out_ref[0:_D, :] = e1_ref[0, :, 0:_D].T * m1
    out_ref[_D:2 * _D, :] = e2_ref[0, :, 0:_D].T * m2

    def mlp(x, w, b, g, be, m, s):
        z = (jnp.log1p(x) - m) / s
        h = jnp.maximum(w[0][:, None] * z[None, :] + b[0][:, None], 0.0)
        mu = h.mean(0, keepdims=True)
        var = ((h - mu) ** 2).mean(0, keepdims=True)
        return (h - mu) * lax.rsqrt(var + 1e-5) * g[0][:, None] + be[0][:, None]

    out_ref[2 * _D:3 * _D, :] = mlp(ck_ref[0, 0, :], wc, bc, gc, bec,
                                    st_ref[0], st_ref[1])
    out_ref[3 * _D:4 * _D, :] = mlp(lk_ref[0, 0, :], wl, bl, gl, bel,
                                    st_ref[2], st_ref[3])
    out_ref[4 * _D:5 * _D, :] = mlp(cm_ref[0, 0, :], wm, bm, gm, bem,
                                    st_ref[4], st_ref[5])


def kernel(category_first, category_second, click_count, like_count, comment_count,
           W_cat1, W_cat2,
           w_click, b_click, g_click, be_click,
           w_like, b_like, g_like, be_like,
           w_comment, b_comment, g_comment, be_comment,
           m_click, s_click, m_like, s_like, m_comment, s_comment):
    idx1 = category_first.astype(jnp.int32)
    idx2 = category_second.astype(jnp.int32)
    w1p = jnp.pad(W_cat1, ((0, 0), (0, 128 - _D)))
    w2p = jnp.pad(W_cat2, ((0, 0), (0, 128 - _D)))

    sc_gather = pl.kernel(
        _sc_gather_body,
        out_type=jax.ShapeDtypeStruct((2, _B, 128), jnp.float32),
        mesh=plsc.VectorSubcoreMesh(core_axis_name="c", subcore_axis_name="s"),
        scratch_types=[
            pltpu.VMEM((_BPW,), jnp.int32),
            pltpu.VMEM((_BPW,), jnp.int32),
            pltpu.VMEM((_BPW, 128), jnp.float32),
            pltpu.SemaphoreType.DMA,
        ],
        compiler_params=pltpu.CompilerParams(use_tc_tiling_on_sc=True),
    )
    emb = sc_gather(w1p, w2p, idx1, idx2)

    stats = jnp.stack([m_click, s_click, m_like, s_like, m_comment, s_comment])

    row_spec = pl.BlockSpec((1, 1, _BN), lambda i: (i, 0, 0))
    par_spec = pl.BlockSpec((1, _D), lambda i: (0, 0))
    out = pl.pallas_call(
        _tc_mlp_body,
        grid=(_NB,),
        in_specs=[
            pl.BlockSpec((1, _BN, 128), lambda i: (0, i, 0)),
            pl.BlockSpec((1, _BN, 128), lambda i: (1, i, 0)),
            row_spec, row_spec, row_spec, row_spec, row_spec,
            par_spec, par_spec, par_spec, par_spec,
            par_spec, par_spec, par_spec, par_spec,
            par_spec, par_spec, par_spec, par_spec,
            pl.BlockSpec(memory_space=pltpu.SMEM),
        ],
        out_specs=pl.BlockSpec((5 * _D, _BN), lambda i: (0, i)),
        out_shape=jax.ShapeDtypeStruct((5 * _D, _B), jnp.float32),
    )(emb, emb,
      idx1.reshape(_NB, 1, _BN), idx2.reshape(_NB, 1, _BN),
      click_count.reshape(_NB, 1, _BN), like_count.reshape(_NB, 1, _BN),
      comment_count.reshape(_NB, 1, _BN),
      w_click, b_click.reshape(1, _D), g_click.reshape(1, _D), be_click.reshape(1, _D),
      w_like, b_like.reshape(1, _D), g_like.reshape(1, _D), be_like.reshape(1, _D),
      w_comment, b_comment.reshape(1, _D), g_comment.reshape(1, _D), be_comment.reshape(1, _D),
      stats)
    return out.T


# speedup vs baseline: 1.3556x; 1.0148x over previous
import jax
import jax.numpy as jnp
from jax import lax
from jax.experimental import pallas as pl
from jax.experimental.pallas import tpu as pltpu
from jax.experimental.pallas import tpu_sc as plsc

_B = 16384
_D = 32
_NC = 2
_NS = 16
_NW = _NC * _NS
_BPW = _B // _NW
_CH = 128
_NCH = _BPW // _CH

_BN = 4096
_NB = _B // _BN


_W2R = 100000


def _sc_gather_body(w1, w2, idx1, idx2, emb, idx1_v, idx2_v, rows_v, sem):
    wid = lax.axis_index("s") * _NC + lax.axis_index("c")
    base = wid * _BPW
    pltpu.sync_copy(idx1.at[pl.ds(base, _BPW)], idx1_v)
    pltpu.sync_copy(idx2.at[pl.ds(base, _BPW)], idx2_v)
    copies = []
    for j in range(_NCH):
        copies.append(
            pltpu.async_copy(w1.at[idx1_v.at[pl.ds(j * _CH, _CH)]],
                             rows_v.at[pl.ds(j * _CH, _CH)], sem))
    for c in copies:
        c.wait()
    pltpu.sync_copy(rows_v, emb.at[0, pl.ds(base, _BPW)])
    copies = []
    for j in range(_NCH):
        copies.append(
            pltpu.async_copy(w2.at[idx2_v.at[pl.ds(j * _CH, _CH)]],
                             rows_v.at[pl.ds(j * _CH, _CH)], sem))
    for c in copies:
        c.wait()
    pltpu.sync_copy(rows_v, emb.at[1, pl.ds(base, _BPW)])


def _tc_mlp_body(e1_ref, e2_ref, c1_ref, c2_ref, ck_ref, lk_ref, cm_ref,
                 wc, bc, gc, bec, wl, bl, gl, bel, wm, bm, gm, bem,
                 st_ref, out_ref):
    m1 = (c1_ref[0, 0, :] != 0).astype(jnp.float32)[None, :]
    m2 = (c2_ref[0, 0, :] != 0).astype(jnp.float32)[None, :]
    out_ref[0:_D, :] = e1_ref[0, :, 0:_D].T * m1
    out_ref[_D:2 * _D, :] = e2_ref[0, :, 0:_D].T * m2

    def mlp(x, w, b, g, be, m, s):
        z = (jnp.log1p(x) - m) / s
        h = jnp.maximum(w[0][:, None] * z[None, :] + b[0][:, None], 0.0)
        mu = h.mean(0, keepdims=True)
        var = ((h - mu) ** 2).mean(0, keepdims=True)
        return (h - mu) * lax.rsqrt(var + 1e-5) * g[0][:, None] + be[0][:, None]

    out_ref[2 * _D:3 * _D, :] = mlp(ck_ref[0, 0, :], wc, bc, gc, bec,
                                    st_ref[0], st_ref[1])
    out_ref[3 * _D:4 * _D, :] = mlp(lk_ref[0, 0, :], wl, bl, gl, bel,
                                    st_ref[2], st_ref[3])
    out_ref[4 * _D:5 * _D, :] = mlp(cm_ref[0, 0, :], wm, bm, gm, bem,
                                    st_ref[4], st_ref[5])


def kernel(category_first, category_second, click_count, like_count, comment_count,
           W_cat1, W_cat2,
           w_click, b_click, g_click, be_click,
           w_like, b_like, g_like, be_like,
           w_comment, b_comment, g_comment, be_comment,
           m_click, s_click, m_like, s_like, m_comment, s_comment):
    idx1 = category_first.astype(jnp.int32)
    idx2 = category_second.astype(jnp.int32)
    w1p = jnp.pad(W_cat1, ((0, 0), (0, 128 - _D)))
    w2p = jnp.pad(W_cat2, ((0, 0), (0, 128 - _D)))

    sc_gather = pl.kernel(
        _sc_gather_body,
        out_type=jax.ShapeDtypeStruct((2, _B, 128), jnp.float32),
        mesh=plsc.VectorSubcoreMesh(core_axis_name="c", subcore_axis_name="s"),
        scratch_types=[
            pltpu.VMEM((_BPW,), jnp.int32),
            pltpu.VMEM((_BPW,), jnp.int32),
            pltpu.VMEM((_BPW, 128), jnp.float32),
            pltpu.SemaphoreType.DMA,
        ],
        compiler_params=pltpu.CompilerParams(use_tc_tiling_on_sc=True),
    )
    emb = sc_gather(w1p, w2p, idx1, idx2)

    stats = jnp.stack([m_click, s_click, m_like, s_like, m_comment, s_comment])

    row_spec = pl.BlockSpec((1, 1, _BN), lambda i: (i, 0, 0))
    par_spec = pl.BlockSpec((1, _D), lambda i: (0, 0))
    out = pl.pallas_call(
        _tc_mlp_body,
        grid=(_NB,),
        in_specs=[
            pl.BlockSpec((1, _BN, 128), lambda i: (0, i, 0)),
            pl.BlockSpec((1, _BN, 128), lambda i: (1, i, 0)),
            row_spec, row_spec, row_spec, row_spec, row_spec,
            par_spec, par_spec, par_spec, par_spec,
            par_spec, par_spec, par_spec, par_spec,
            par_spec, par_spec, par_spec, par_spec,
            pl.BlockSpec(memory_space=pltpu.SMEM),
        ],
        out_specs=pl.BlockSpec((5 * _D, _BN), lambda i: (0, i)),
        out_shape=jax.ShapeDtypeStruct((5 * _D, _B), jnp.float32),
    )(emb, emb,
      idx1.reshape(_NB, 1, _BN), idx2.reshape(_NB, 1, _BN),
      click_count.reshape(_NB, 1, _BN), like_count.reshape(_NB, 1, _BN),
      comment_count.reshape(_NB, 1, _BN),
      w_click, b_click.reshape(1, _D), g_click.reshape(1, _D), be_click.reshape(1, _D),
      w_like, b_like.reshape(1, _D), g_like.reshape(1, _D), be_like.reshape(1, _D),
      w_comment, b_comment.reshape(1, _D), g_comment.reshape(1, _D), be_comment.reshape(1, _D),
      stats)
    return out.T
